# baseline (speedup 1.0000x reference)
import functools

import jax
import jax.numpy as jnp
from jax import lax
from jax.experimental import pallas as pl
from jax.experimental.pallas import tpu as pltpu
from jax.experimental.pallas import tpu_sc as plsc

EMBED_DIM = 64
BATCH = 4096
SEQ = 200
NC = 2
NS = 16
NW = NC * NS
ROWS_PER_W = BATCH // NW
NSLOT = 4
NGROUPS = ROWS_PER_W // NSLOT

_mesh = plsc.VectorSubcoreMesh(core_axis_name="c", subcore_axis_name="s")


@functools.partial(
    pl.kernel,
    out_type=jax.ShapeDtypeStruct((BATCH, SEQ, EMBED_DIM), jnp.float32),
    mesh=_mesh,
    scratch_types=[
        pltpu.VMEM((ROWS_PER_W, SEQ), jnp.int32),
        pltpu.VMEM((NSLOT, SEQ, EMBED_DIM), jnp.float32),
    ]
    + [pltpu.SemaphoreType.DMA] * (2 * NSLOT),
    compiler_params=pltpu.CompilerParams(use_tc_tiling_on_sc=False),
)
def _embed_gather(table_hbm, x_hbm, out_hbm, idx_v, rows_v, *sems):
    gsem = sems[:NSLOT]
    wsem = sems[NSLOT:]
    wid = lax.axis_index("s") * NC + lax.axis_index("c")
    b0 = wid * ROWS_PER_W
    pltpu.sync_copy(x_hbm.at[pl.ds(b0, ROWS_PER_W)], idx_v)

    for s in range(NSLOT):
        pltpu.async_copy(table_hbm.at[idx_v.at[s]], rows_v.at[s], gsem[s])

    @pl.loop(0, NGROUPS)
    def _ring(grp):
        i0 = grp * NSLOT
        for s in range(NSLOT):
            pltpu.make_async_copy(
                table_hbm.at[idx_v.at[i0 + s]], rows_v.at[s], gsem[s]
            ).wait()
            pltpu.async_copy(rows_v.at[s], out_hbm.at[b0 + i0 + s], wsem[s])
        for s in range(NSLOT):
            pltpu.make_async_copy(
                rows_v.at[s], out_hbm.at[b0 + i0 + s], wsem[s]
            ).wait()

            @pl.when(grp < NGROUPS - 1)
            def _():
                pltpu.async_copy(
                    table_hbm.at[idx_v.at[i0 + NSLOT + s]], rows_v.at[s], gsem[s]
                )


def kernel(x, table):
    return _embed_gather(table, x.astype(jnp.int32))
